# baseline (speedup 1.0000x reference)
import functools

import jax
import jax.numpy as jnp
from jax import lax
from jax.experimental import pallas as pl
from jax.experimental.pallas import tpu as pltpu
from jax.experimental.pallas import tpu_sc as plsc


def _gather_kernel(B, T, D):
    info = plsc.get_sparse_core_info()
    NC, NS = info.num_cores, info.num_subcores
    NW = NC * NS
    NBUF = 2
    assert B % NW == 0
    NB = B // NW
    LG = 16

    mesh = plsc.VectorSubcoreMesh(core_axis_name="c", subcore_axis_name="s")

    @functools.partial(
        pl.kernel,
        out_type=jax.ShapeDtypeStruct((T, D, B), jnp.float32),
        mesh=mesh,
        scratch_types=[
            pltpu.VMEM((NB * T,), jnp.int32),
            pltpu.VMEM((T * NB,), jnp.int32),
            [pltpu.VMEM((NB, 128), jnp.float32) for _ in range(NBUF)],
            [pltpu.VMEM((D, NB), jnp.float32) for _ in range(NBUF)],
            [pltpu.SemaphoreType.DMA for _ in range(NBUF)],
            [pltpu.SemaphoreType.DMA for _ in range(NBUF)],
        ],
        compiler_params=pltpu.CompilerParams(
            use_tc_tiling_on_sc=True, needs_layout_passes=False
        ),
    )
    def k(idx_hbm, table_hbm, out_hbm, idx_v, idx_t, rows_g, rows_t, sem_g, sem_s):
        wid = lax.axis_index("s") * NC + lax.axis_index("c")
        b0 = wid * NB

        pltpu.sync_copy(idx_hbm.at[pl.ds(b0 * T, NB * T)], idx_v)

        bases = [lax.iota(jnp.int32, LG) * T + (g * LG * T) for g in range(NB // LG)]

        @pl.loop(0, T)
        def _(t):
            for g in range(NB // LG):
                vals = plsc.load_gather(idx_v, [bases[g] + t])
                idx_t[pl.ds(t * NB + g * LG, LG)] = vals

        def start_gather(t, b):
            pltpu.async_copy(
                table_hbm.at[idx_t.at[pl.ds(t * NB, NB)]], rows_g[b], sem_g[b]
            )

        def wait_gather(t, b):
            pltpu.make_async_copy(
                table_hbm.at[idx_t.at[pl.ds(t * NB, NB)]], rows_g[b], sem_g[b]
            ).wait()

        def store(t, b, wait):
            dst = out_hbm.at[t, :, pl.ds(b0, NB)]
            if wait:
                pltpu.make_async_copy(rows_t[b], dst, sem_s[b]).wait()
            else:
                pltpu.async_copy(rows_t[b], dst, sem_s[b])

        tokvecs = [lax.iota(jnp.int32, LG) + g * LG for g in range(NB // LG)]

        def transpose(b):
            g_ref, t_ref = rows_g[b], rows_t[b]

            @pl.loop(0, D)
            def _(d):
                dvec = jnp.full((LG,), 0, jnp.int32) + d
                for g in range(NB // LG):
                    vals = plsc.load_gather(g_ref, [tokvecs[g], dvec])
                    t_ref[d, pl.ds(g * LG, LG)] = vals

        for b in range(NBUF):
            start_gather(b, b)

        @pl.loop(0, T, step=NBUF)
        def _(g):
            for b in range(NBUF):
                t = g + b
                wait_gather(t, b)

                @pl.when(t >= NBUF)
                def _():
                    store(t - NBUF, b, wait=True)

                transpose(b)
                store(t, b, wait=False)

                @pl.when(t + NBUF < T)
                def _():
                    start_gather(t + NBUF, b)

        for b in range(NBUF):
            store(T - NBUF + b, b, wait=True)

    return k


def kernel(tokens, embedding_weight):
    B, T = tokens.shape
    V, D = embedding_weight.shape
    k = _gather_kernel(B, T, D)
    flat_idx = tokens.reshape(B * T).astype(jnp.int32)
    table_p = jnp.pad(embedding_weight, ((0, 0), (0, 128 - D)))
    out_t = k(flat_idx, table_p)
    return jnp.transpose(out_t, (2, 0, 1))
